# baseline (speedup 1.0000x reference)
import functools

import jax
import jax.numpy as jnp
from jax import lax
from jax.experimental import pallas as pl
from jax.experimental.pallas import tpu as pltpu
from jax.experimental.pallas import tpu_sc as plsc

K = 2
TM = 512
NH = 8
SC_CH = 96


def _sc_row_gather(table, idx):
    _, d = table.shape
    b = idx.shape[0]
    info = plsc.get_sparse_core_info()
    nw = info.num_cores * info.num_subcores
    bpw = b // nw
    pieces = []
    off = 0
    while off < bpw:
        sz = min(SC_CH, bpw - off)
        pieces.append((off, sz))
        off += sz
    mesh = plsc.VectorSubcoreMesh(core_axis_name="c", subcore_axis_name="s")

    @functools.partial(
        pl.kernel, mesh=mesh,
        out_type=jax.ShapeDtypeStruct((b, d), table.dtype),
        scratch_types=[
            pltpu.VMEM((bpw,), jnp.int32),
            pltpu.VMEM((SC_CH, d), table.dtype),
            pltpu.SemaphoreType.DMA,
        ],
    )
    def k(table_hbm, idx_hbm, out_hbm, idx_v, rows_v, sem):
        wid = lax.axis_index("s") * info.num_cores + lax.axis_index("c")
        base = wid * bpw
        pltpu.sync_copy(idx_hbm.at[pl.ds(base, bpw)], idx_v)
        for off, sz in pieces:
            pltpu.async_copy(table_hbm.at[idx_v.at[pl.ds(off, sz)]],
                             rows_v.at[pl.ds(0, sz)], sem).wait()
            pltpu.sync_copy(rows_v.at[pl.ds(0, sz)],
                            out_hbm.at[pl.ds(base + off, sz)])

    return k(table, idx)


def _gemm_body(eid_ref, rs_ref, nv_ref, xs_ref, p_ref,
               w1_ref, w3_ref, w2_ref, ys_ref, *, nh):
    h = pl.program_id(0)
    w = pl.program_id(1)
    nv = nv_ref[w]
    rs = pl.multiple_of(rs_ref[w], 8)

    @pl.when(nv > 0)
    def _():
        xb = xs_ref[pl.ds(rs, TM), :].astype(jnp.bfloat16)
        w1b = w1_ref[0].astype(jnp.bfloat16)
        w3b = w3_ref[0].astype(jnp.bfloat16)
        w2b = w2_ref[0].astype(jnp.bfloat16)
        g = jnp.dot(xb, w1b, preferred_element_type=jnp.float32)
        u = jnp.dot(xb, w3b, preferred_element_type=jnp.float32)
        hh = (g * jax.nn.sigmoid(g) * u).astype(jnp.bfloat16)
        part = jnp.dot(hh, w2b, preferred_element_type=jnp.float32)

        mask = jax.lax.broadcasted_iota(jnp.int32, (TM, 1), 0) < nv
        old = ys_ref[pl.ds(rs, TM), :]

        @pl.when(h == 0)
        def _():
            ys_ref[pl.ds(rs, TM), :] = jnp.where(mask, part, old)

        @pl.when(jnp.logical_and(h > 0, h < nh - 1))
        def _():
            ys_ref[pl.ds(rs, TM), :] = jnp.where(mask, old + part, old)

        @pl.when(h == nh - 1)
        def _():
            p_blk = p_ref[pl.ds(rs, TM), :]
            ys_ref[pl.ds(rs, TM), :] = jnp.where(mask, (old + part) * p_blk, old)


def _pair_add_body(za_ref, zb_ref, out_ref):
    out_ref[...] = za_ref[...] + zb_ref[...]


def kernel(x, gate_w, w1, w3, w2):
    b, s, d = x.shape
    e_num, _, hid = w1.shape
    t_num = b * s
    r_num = t_num * K
    r_pad = ((r_num + 8 * (e_num - 1) + TM - 8 + 255) // 256) * 256
    xf = x.reshape(t_num, d)

    scores = xf @ gate_w.T
    expert_weights, expert_indices = jax.lax.top_k(scores, K)
    expert_weights = jax.nn.softmax(expert_weights, axis=-1)

    ef = expert_indices.reshape(-1).astype(jnp.int32)
    order = jnp.argsort(ef).astype(jnp.int32)
    ef_s = ef[order]
    tok = (order // K).astype(jnp.int32)
    p_sorted = expert_weights.reshape(-1)[order]
    counts = jnp.bincount(ef, length=e_num).astype(jnp.int32)
    starts = (jnp.cumsum(counts) - counts).astype(jnp.int32)
    pc = ((counts + 7) // 8) * 8
    pstart = (jnp.cumsum(pc) - pc).astype(jnp.int32)
    ppos = (jnp.arange(r_num, dtype=jnp.int32) - starts[ef_s] + pstart[ef_s])

    tok2 = jnp.zeros((r_pad,), jnp.int32).at[ppos].set(tok)
    p2 = jnp.zeros((r_pad,), jnp.float32).at[ppos].set(p_sorted)[:, None]
    inv2 = jnp.zeros((r_num,), jnp.int32).at[order].set(ppos)
    inv3 = jnp.concatenate([inv2[0::2], inv2[1::2]])

    maxj = r_num // TM
    w_items = e_num + maxj - 1
    nch = (pc + TM - 1) // TM
    e_c = jnp.repeat(jnp.arange(e_num, dtype=jnp.int32), maxj)
    j_c = jnp.tile(jnp.arange(maxj, dtype=jnp.int32), e_num)
    validc = j_c < nch[e_c]
    ordc = jnp.argsort(jnp.logical_not(validc).astype(jnp.int32))[:w_items]
    v_w = validc[ordc]
    eidw = jnp.where(v_w, e_c[ordc], e_num - 1).astype(jnp.int32)
    rsw = jnp.where(v_w, pstart[e_c[ordc]] + j_c[ordc] * TM, 0).astype(jnp.int32)
    nvw = (jnp.clip((pstart + counts)[e_c[ordc]] - rsw, 0, TM)
           * v_w).astype(jnp.int32)

    xs = _sc_row_gather(xf, tok2)

    hb = hid // NH
    grid_spec = pltpu.PrefetchScalarGridSpec(
        num_scalar_prefetch=3,
        grid=(NH, w_items),
        in_specs=[
            pl.BlockSpec((r_pad, d), lambda h, w, eid, rs, nv: (0, 0)),
            pl.BlockSpec((r_pad, 1), lambda h, w, eid, rs, nv: (0, 0)),
            pl.BlockSpec((1, d, hb), lambda h, w, eid, rs, nv: (eid[w], 0, h)),
            pl.BlockSpec((1, d, hb), lambda h, w, eid, rs, nv: (eid[w], 0, h)),
            pl.BlockSpec((1, hb, d), lambda h, w, eid, rs, nv: (eid[w], h, 0)),
        ],
        out_specs=pl.BlockSpec((r_pad, d), lambda h, w, eid, rs, nv: (0, 0)),
    )
    ys = pl.pallas_call(
        functools.partial(_gemm_body, nh=NH),
        grid_spec=grid_spec,
        out_shape=jax.ShapeDtypeStruct((r_pad, d), jnp.float32),
        compiler_params=pltpu.CompilerParams(
            dimension_semantics=("arbitrary", "arbitrary"),
            vmem_limit_bytes=100 * 1024 * 1024,
        ),
    )(eidw, rsw, nvw, xs, p2, w1, w3, w2)

    z = _sc_row_gather(ys, inv3)

    nt = t_num // TM
    out = pl.pallas_call(
        _pair_add_body,
        grid=(nt,),
        in_specs=[
            pl.BlockSpec((TM, d), lambda i: (i, 0)),
            pl.BlockSpec((TM, d), lambda i: (i + nt, 0)),
        ],
        out_specs=pl.BlockSpec((TM, d), lambda i: (i, 0)),
        out_shape=jax.ShapeDtypeStruct((t_num, d), jnp.float32),
    )(z, z)

    return out.reshape(b, s, d)
